# baseline (speedup 1.0000x reference)
import functools

import jax
import jax.numpy as jnp
from jax import lax
from jax.experimental import pallas as pl
from jax.experimental.pallas import tpu as pltpu
from jax.experimental.pallas import tpu_sc as plsc

HEAD_DIM = 256
_HALF = HEAD_DIM // 2
B_TOTAL = 4 * 8192

_info = plsc.get_sparse_core_info()
_NC, _NS = _info.num_cores, _info.num_subcores
_NW = _NC * _NS
_B_PER_W = B_TOTAL // _NW
_CHUNK = 64
_NCHUNK = _B_PER_W // _CHUNK
_SG = 4
_SW = 4
_LAG = 3


def _rope_gather(pos2_flat, cos_half, sin_half):
    mesh = plsc.VectorSubcoreMesh(core_axis_name="c", subcore_axis_name="s")

    @functools.partial(
        pl.kernel,
        mesh=mesh,
        out_type=[
            jax.ShapeDtypeStruct((B_TOTAL, HEAD_DIM), jnp.float32),
            jax.ShapeDtypeStruct((B_TOTAL, HEAD_DIM), jnp.float32),
        ],
        scratch_types=[
            pltpu.VMEM((_B_PER_W,), jnp.int32),
        ]
        + [pltpu.VMEM((_CHUNK, _HALF), jnp.float32)] * _SG
        + [pltpu.VMEM((_CHUNK, HEAD_DIM), jnp.float32)] * _SW
        + [pltpu.SemaphoreType.DMA] * (_SG + _SW),
    )
    def k(pos_hbm, cos_hbm, sin_hbm, outc_hbm, outs_hbm, idx_v, *rest):
        hbufs = list(rest[:_SG])
        fbufs = list(rest[_SG:_SG + _SW])
        gsem = list(rest[_SG + _SW:2 * _SG + _SW])
        wsem = list(rest[2 * _SG + _SW:])
        wid = lax.axis_index("s") * _NC + lax.axis_index("c")
        base = wid * _B_PER_W
        pltpu.sync_copy(pos_hbm.at[pl.ds(base, _B_PER_W)], idx_v)

        chunks = []
        for j in range(_NCHUNK):
            chunks.append((cos_hbm, outc_hbm, j))
            chunks.append((sin_hbm, outs_hbm, j))
        m = len(chunks)

        def dup(hb, fb):
            def row(r, _):
                for c in range(_HALF // 16):
                    v = hb[r, pl.ds(c * 16, 16)]
                    fb[r, pl.ds(c * 16, 16)] = v
                    fb[r, pl.ds(_HALF + c * 16, 16)] = v
                return 0

            lax.fori_loop(0, _CHUNK, row, 0, unroll=2)

        gh = [None] * _SG
        wh = [None] * _SW
        for t in range(m + _LAG):
            if t < m:
                b = t % _SG
                tbl, _, j = chunks[t]
                idxs = idx_v.at[pl.ds(j * _CHUNK, _CHUNK)]
                gh[b] = pltpu.async_copy(tbl.at[idxs], hbufs[b], gsem[b])
            tt = t - _LAG
            if tt >= 0:
                gb = tt % _SG
                fb = tt % _SW
                gh[gb].wait()
                if tt >= _SW:
                    wh[fb].wait()
                dup(hbufs[gb], fbufs[fb])
                _, out, j = chunks[tt]
                row0 = base + j * _CHUNK
                wh[fb] = pltpu.async_copy(
                    fbufs[fb], out.at[pl.ds(row0, _CHUNK)], wsem[fb])
        for t in range(m - _SW, m):
            wh[t % _SW].wait()

    return k(pos2_flat, cos_half, sin_half)


def kernel(x, position_ids, cos_cached, sin_cached):
    b, s = position_ids.shape
    pos2_flat = position_ids.reshape(-1) * 2
    cos_half = cos_cached.reshape(-1, _HALF)
    sin_half = sin_cached.reshape(-1, _HALF)
    cos, sin = _rope_gather(pos2_flat, cos_half, sin_half)
    return (cos.reshape(b, s, HEAD_DIM).astype(x.dtype),
            sin.reshape(b, s, HEAD_DIM).astype(x.dtype))


# speedup vs baseline: 4.9788x; 4.9788x over previous
import functools

import jax
import jax.numpy as jnp
from jax import lax
from jax.experimental import pallas as pl
from jax.experimental.pallas import tpu as pltpu
from jax.experimental.pallas import tpu_sc as plsc

HEAD_DIM = 256
B_TOTAL = 4 * 8192

_info = plsc.get_sparse_core_info()
_NC, _NS = _info.num_cores, _info.num_subcores
_NW = _NC * _NS
_B_PER_W = B_TOTAL // _NW
_CHUNK = 128
_NCHUNK = _B_PER_W // _CHUNK
_DEPTH = 3


def _rope_gather(pos_flat, cos_cached, sin_cached):
    mesh = plsc.VectorSubcoreMesh(core_axis_name="c", subcore_axis_name="s")

    @functools.partial(
        pl.kernel,
        mesh=mesh,
        out_type=[
            jax.ShapeDtypeStruct((B_TOTAL, HEAD_DIM), jnp.float32),
            jax.ShapeDtypeStruct((B_TOTAL, HEAD_DIM), jnp.float32),
        ],
        scratch_types=[
            pltpu.VMEM((_B_PER_W,), jnp.int32),
        ]
        + [pltpu.VMEM((_CHUNK, HEAD_DIM), jnp.float32)] * _DEPTH
        + [pltpu.SemaphoreType.DMA] * (2 * _DEPTH),
    )
    def k(pos_hbm, cos_hbm, sin_hbm, outc_hbm, outs_hbm, idx_v, *rest):
        bufs = list(rest[:_DEPTH])
        gsem = list(rest[_DEPTH:2 * _DEPTH])
        wsem = list(rest[2 * _DEPTH:])
        wid = lax.axis_index("s") * _NC + lax.axis_index("c")
        base = wid * _B_PER_W
        pltpu.sync_copy(pos_hbm.at[pl.ds(base, _B_PER_W)], idx_v)

        chunks = []
        for j in range(_NCHUNK):
            chunks.append((cos_hbm, outc_hbm, j))
            chunks.append((sin_hbm, outs_hbm, j))
        m = len(chunks)

        gh = [None] * _DEPTH
        wh = [None] * _DEPTH
        lag = _DEPTH - 1
        for t in range(m + lag):
            if t < m:
                b = t % _DEPTH
                if t >= _DEPTH:
                    wh[b].wait()
                tbl, _, j = chunks[t]
                idxs = idx_v.at[pl.ds(j * _CHUNK, _CHUNK)]
                gh[b] = pltpu.async_copy(tbl.at[idxs], bufs[b], gsem[b])
            tt = t - lag
            if tt >= 0:
                tb = tt % _DEPTH
                gh[tb].wait()
                _, out, j = chunks[tt]
                row0 = base + j * _CHUNK
                wh[tb] = pltpu.async_copy(
                    bufs[tb], out.at[pl.ds(row0, _CHUNK)], wsem[tb])
        for t in range(m - _DEPTH, m):
            wh[t % _DEPTH].wait()

    return k(pos_flat, cos_cached, sin_cached)


def kernel(x, position_ids, cos_cached, sin_cached):
    b, s = position_ids.shape
    pos_flat = position_ids.reshape(-1)
    cos, sin = _rope_gather(pos_flat, cos_cached, sin_cached)
    return (cos.reshape(b, s, HEAD_DIM).astype(x.dtype),
            sin.reshape(b, s, HEAD_DIM).astype(x.dtype))
